# baseline (speedup 1.0000x reference)
import jax
import jax.numpy as jnp
from jax.experimental import pallas as pl
from jax.experimental.pallas import tpu as pltpu

_ROWS = 1024
_COLS = 27278
_PAR = 2
_BLOCK_ROWS = 32
_STEPS = _ROWS // _PAR // _BLOCK_ROWS


def _masked_mse_body(o_ref, t_ref, res_ref, acc_ref, cnt_ref):
    i = pl.program_id(1)

    @pl.when(i == 0)
    def _init():
        acc_ref[0] = 0.0
        cnt_ref[0] = 0.0

    o = o_ref[...]
    t = t_ref[...]
    m = t != -1.0
    d = o - t
    acc_ref[0] += jnp.sum(jnp.where(m, d * d, 0.0))
    cnt_ref[0] += jnp.sum(m.astype(jnp.float32))

    @pl.when(i == _STEPS - 1)
    def _fin():
        res_ref[0, 0, 0] = acc_ref[0]
        res_ref[0, 0, 1] = cnt_ref[0]


def kernel(output, target):
    spec = pl.BlockSpec((_BLOCK_ROWS, _COLS), lambda p, i: (p * _STEPS + i, 0))
    partials = pl.pallas_call(
        _masked_mse_body,
        grid=(_PAR, _STEPS),
        in_specs=[spec, spec],
        out_specs=pl.BlockSpec((1, 1, 2), lambda p, i: (p, 0, 0), memory_space=pltpu.SMEM),
        out_shape=jax.ShapeDtypeStruct((_PAR, 1, 2), jnp.float32),
        scratch_shapes=[
            pltpu.SMEM((1,), jnp.float32),
            pltpu.SMEM((1,), jnp.float32),
        ],
        compiler_params=pltpu.CompilerParams(
            dimension_semantics=("parallel", "arbitrary"),
        ),
    )(output, target)
    return partials[:, 0, 0].sum() / partials[:, 0, 1].sum()


# speedup vs baseline: 1.0108x; 1.0108x over previous
import jax
import jax.numpy as jnp
from jax.experimental import pallas as pl
from jax.experimental.pallas import tpu as pltpu

_ROWS = 1024
_COLS = 27278
_BLOCK_ROWS = 32
_STEPS = _ROWS // _BLOCK_ROWS


def _masked_mse_body(o_ref, t_ref, res_ref, acc_ref, cnt_ref):
    i = pl.program_id(0)

    @pl.when(i == 0)
    def _init():
        acc_ref[...] = jnp.zeros_like(acc_ref)
        cnt_ref[...] = jnp.zeros_like(cnt_ref)

    o = o_ref[...]
    t = t_ref[...]
    m = t != -1.0
    d = o - t
    acc_ref[...] += jnp.where(m, d * d, 0.0)
    cnt_ref[...] += m.astype(jnp.float32)

    @pl.when(i == _STEPS - 1)
    def _fin():
        res_ref[0, 0] = jnp.sum(acc_ref[...]) / jnp.sum(cnt_ref[...])


def kernel(output, target):
    spec = pl.BlockSpec((_BLOCK_ROWS, _COLS), lambda i: (i, 0))
    res = pl.pallas_call(
        _masked_mse_body,
        grid=(_STEPS,),
        in_specs=[spec, spec],
        out_specs=pl.BlockSpec(memory_space=pltpu.SMEM),
        out_shape=jax.ShapeDtypeStruct((1, 1), jnp.float32),
        scratch_shapes=[
            pltpu.VMEM((_BLOCK_ROWS, _COLS), jnp.float32),
            pltpu.VMEM((_BLOCK_ROWS, _COLS), jnp.float32),
        ],
    )(output, target)
    return res.reshape(())
